# baseline (speedup 1.0000x reference)
import functools

import jax
import jax.numpy as jnp
from jax import lax
from jax.experimental import pallas as pl
from jax.experimental.pallas import tpu as pltpu
from jax.experimental.pallas import tpu_sc as plsc

_N = 10000
_D = 128
_NC = 16
_BLK = 2000

_E = 320000
_NTILES = 32
_NW = _D // 2
_WPT = 4
_NSLOT = 8
_EC = 3200
_NCHUNK = _E // _EC
_CPH = _NCHUNK // 2
_GRP = _EC // 16
_BOTH_GE = 0x3F803F80


def _stage_a_body(mask_ref, x_ref, tok_ref, Wp_ref, bp_ref, Ws_ref,
                  pool_ref, self_ref):
    xm = jnp.where(mask_ref[...] > 0, tok_ref[...], x_ref[...])
    pool_ref[...] = jnp.maximum(
        jnp.dot(xm, Wp_ref[...], preferred_element_type=jnp.float32)
        + bp_ref[...], 0.0)
    self_ref[...] = jnp.dot(xm, Ws_ref[...], preferred_element_type=jnp.float32)


def _stage_a(mask, x, tok, Wp, bp, Ws):
    grid = _N // _BLK
    row_spec = pl.BlockSpec((_BLK, _D), lambda i: (i, 0))
    full_mat = pl.BlockSpec((_D, _D), lambda i: (0, 0))
    full_vec = pl.BlockSpec((1, _D), lambda i: (0, 0))
    return pl.pallas_call(
        _stage_a_body,
        grid=(grid,),
        in_specs=[pl.BlockSpec((_BLK, 1), lambda i: (i, 0)),
                  row_spec, full_vec, full_mat, full_vec, full_mat],
        out_specs=[row_spec, row_spec],
        out_shape=[jax.ShapeDtypeStruct((_N, _D), jnp.float32),
                   jax.ShapeDtypeStruct((_N, _D), jnp.float32)],
    )(mask, x, tok, Wp, bp, Ws)


def _stage_b_body(self1_ref, neigh1_ref, Wn1_ref, b1_ref, Wnp_ref, bnp_ref,
                  gamma_ref, beta_ref, We2d_ref, Wp2_ref, bp2_ref, Ws2_ref,
                  ns_ref, pool2_ref, self2_ref):
    enc = jnp.maximum(
        self1_ref[...]
        + jnp.dot(neigh1_ref[...], Wn1_ref[...],
                  preferred_element_type=jnp.float32)
        + b1_ref[...], 0.0)
    z = jnp.dot(enc, Wnp_ref[...], preferred_element_type=jnp.float32) \
        + bnp_ref[...]
    mu = jnp.mean(z, axis=-1, keepdims=True)
    var = jnp.mean((z - mu) ** 2, axis=-1, keepdims=True)
    ns_ref[...] = (z - mu) / jnp.sqrt(var + 1e-5) * gamma_ref[...] \
        + beta_ref[...]
    rep = jnp.dot(enc, We2d_ref[...], preferred_element_type=jnp.float32)
    pool2_ref[...] = jnp.maximum(
        jnp.dot(rep, Wp2_ref[...], preferred_element_type=jnp.float32)
        + bp2_ref[...], 0.0)
    self2_ref[...] = jnp.dot(rep, Ws2_ref[...],
                             preferred_element_type=jnp.float32)


def _stage_b(self1, neigh1, Wn1, b1, Wnp, bnp, gamma, beta, We2d, Wp2, bp2,
             Ws2):
    grid = _N // _BLK
    row_spec = pl.BlockSpec((_BLK, _D), lambda i: (i, 0))
    full_mat = pl.BlockSpec((_D, _D), lambda i: (0, 0))
    full_vec = pl.BlockSpec((1, _D), lambda i: (0, 0))
    nc_mat = pl.BlockSpec((_D, _NC), lambda i: (0, 0))
    nc_vec = pl.BlockSpec((1, _NC), lambda i: (0, 0))
    return pl.pallas_call(
        _stage_b_body,
        grid=(grid,),
        in_specs=[row_spec, row_spec, full_mat, full_vec, nc_mat, nc_vec,
                  nc_vec, nc_vec, full_mat, full_mat, full_vec, full_mat],
        out_specs=[pl.BlockSpec((_BLK, _NC), lambda i: (i, 0)),
                   row_spec, row_spec],
        out_shape=[jax.ShapeDtypeStruct((_N, _NC), jnp.float32),
                   jax.ShapeDtypeStruct((_N, _D), jnp.float32),
                   jax.ShapeDtypeStruct((_N, _D), jnp.float32)],
    )(self1, neigh1, Wn1, b1, Wnp, bnp, gamma, beta, We2d, Wp2, bp2, Ws2)


def _stage_c_body(self2_ref, neigh2_ref, Wn2_ref, b2_ref, recon_ref):
    recon_ref[...] = jnp.maximum(
        self2_ref[...]
        + jnp.dot(neigh2_ref[...], Wn2_ref[...],
                  preferred_element_type=jnp.float32)
        + b2_ref[...], 0.0)


def _stage_c(self2, neigh2, Wn2, b2):
    grid = _N // _BLK
    row_spec = pl.BlockSpec((_BLK, _D), lambda i: (i, 0))
    return pl.pallas_call(
        _stage_c_body,
        grid=(grid,),
        in_specs=[row_spec, row_spec,
                  pl.BlockSpec((_D, _D), lambda i: (0, 0)),
                  pl.BlockSpec((1, _D), lambda i: (0, 0))],
        out_specs=row_spec,
        out_shape=jax.ShapeDtypeStruct((_N, _D), jnp.float32),
    )(self2, neigh2, Wn2, b2)


_UNROLL = 8


def _sc_segmax_body(poolP_hbm, edata_hbm, out_hbm,
                    p0, p1, p2, p3, a0, a1, a2, a3,
                    ebuf0, ebuf1, chkb, sem0, sem1):
    pools = [p0, p1, p2, p3]
    accs = [a0, a1, a2, a3]
    ebufs = [ebuf0, ebuf1]
    sems = [sem0, sem1]
    core = lax.axis_index("c")
    sub = lax.axis_index("s")
    half = sub // _NSLOT
    fslot = sub % _NSLOT
    wbase = core * (_NSLOT * _WPT) + fslot * _WPT
    cbase = half * _CPH

    pltpu.async_copy(edata_hbm.at[cbase], ebuf0, sem0)
    pltpu.async_copy(edata_hbm.at[cbase + 1], ebuf1, sem1)

    for f in range(_WPT):
        pltpu.sync_copy(poolP_hbm.at[wbase + f], pools[f])

    zeros16 = jnp.zeros((16,), jnp.int32)

    def zbody(i, carry):
        for f in range(_WPT):
            accs[f][pl.ds(i * 16, 16)] = zeros16
        return carry

    lax.fori_loop(0, _N // 16, zbody, 0)

    def do_chunk(b):
        ebuf = ebufs[b]

        def grp_body(g, gcarry):
            iota = lax.iota(jnp.int32, 16)
            d16s, valss = [], []
            bad = None
            for j in range(_UNROLL):
                e = (g * _UNROLL + j) * 16
                s16 = ebuf[pl.ds(e, 16)]
                d16 = ebuf[pl.ds(_EC + e, 16)]
                w_bf = plsc.bitcast(ebuf[pl.ds(2 * _EC + e, 16)],
                                    jnp.bfloat16)
                vals = [plsc.bitcast(plsc.load_gather(pools[f], [s16]),
                                     jnp.bfloat16) * w_bf
                        for f in range(_WPT)]
                plsc.store_scatter(chkb, [d16], iota)
                rd = plsc.load_gather(chkb, [d16])
                miss = rd != iota
                bad = miss if bad is None else jnp.logical_or(bad, miss)
                d16s.append(d16)
                valss.append(vals)
            nodup = jnp.logical_not(jnp.any(bad))

            def fast(_):
                for j in range(_UNROLL):
                    for f in range(_WPT):
                        old = plsc.load_gather(accs[f], [d16s[j]])
                        new_bf = jnp.maximum(
                            plsc.bitcast(old, jnp.bfloat16), valss[j][f])
                        plsc.store_scatter(accs[f], [d16s[j]],
                                           plsc.bitcast(new_bf, jnp.int32))
                return 0

            def slow(_):
                pends = tuple([jnp.ones((16,), jnp.int32)] *
                              (_UNROLL * _WPT))

                def wcond(p):
                    anyp = p[0]
                    for q in p[1:]:
                        anyp = anyp | q
                    return jnp.any(anyp != 0)

                def wbody(p):
                    npend = []
                    for j in range(_UNROLL):
                        for f in range(_WPT):
                            k = j * _WPT + f
                            val = valss[j][f]
                            old = plsc.load_gather(accs[f], [d16s[j]])
                            new_bf = jnp.maximum(
                                plsc.bitcast(old, jnp.bfloat16), val)
                            plsc.store_scatter(
                                accs[f], [d16s[j]],
                                plsc.bitcast(new_bf, jnp.int32),
                                mask=(p[k] != 0))
                            chk = plsc.load_gather(accs[f], [d16s[j]])
                            ge = plsc.bitcast(chk, jnp.bfloat16) >= val
                            flag = plsc.bitcast(
                                jnp.where(ge, jnp.bfloat16(1.0),
                                          jnp.bfloat16(0.0)), jnp.int32)
                            npend.append(
                                (flag != _BOTH_GE).astype(jnp.int32))
                    return tuple(npend)

                lax.while_loop(wcond, wbody, pends)
                return 0

            lax.cond(nodup, fast, slow, 0)
            return gcarry

        lax.fori_loop(0, _GRP // _UNROLL, grp_body, 0)

    def chunk_pair(k, carry):
        for b in range(2):
            c = cbase + k * 2 + b
            pltpu.make_async_copy(edata_hbm.at[0], ebufs[b], sems[b]).wait()
            do_chunk(b)
            nxt = jnp.minimum(c + 2, cbase + _CPH - 1)
            pltpu.async_copy(edata_hbm.at[nxt], ebufs[b], sems[b])
        return carry

    lax.fori_loop(0, _CPH // 2, chunk_pair, 0)
    for b in range(2):
        pltpu.make_async_copy(edata_hbm.at[0], ebufs[b], sems[b]).wait()

    for f in range(_WPT):
        pltpu.sync_copy(accs[f], out_hbm.at[half * _NW + wbase + f])


def _segmax(pool, src, dst, ew):
    poolP = lax.bitcast_convert_type(
        pool.astype(jnp.bfloat16).reshape(_N, _NW, 2), jnp.int32).T
    ewb = ew.astype(jnp.bfloat16)
    ewP = lax.bitcast_convert_type(jnp.stack([ewb, ewb], axis=-1), jnp.int32)
    edata = jnp.stack([src, dst, ewP]).reshape(3, _NCHUNK, _EC) \
        .transpose(1, 0, 2).reshape(_NCHUNK, 3 * _EC)

    mesh = plsc.VectorSubcoreMesh(core_axis_name="c", subcore_axis_name="s")
    kern = functools.partial(
        pl.kernel,
        mesh=mesh,
        compiler_params=pltpu.CompilerParams(needs_layout_passes=False),
        out_type=jax.ShapeDtypeStruct((2 * _NW, _N), jnp.int32),
        scratch_types=(
            [pltpu.VMEM((_N,), jnp.int32) for _ in range(2 * _WPT)]
            + [pltpu.VMEM((3 * _EC,), jnp.int32),
               pltpu.VMEM((3 * _EC,), jnp.int32),
               pltpu.VMEM((_N,), jnp.int32),
               pltpu.SemaphoreType.DMA, pltpu.SemaphoreType.DMA]
        ),
    )(_sc_segmax_body)
    out2 = lax.bitcast_convert_type(kern(poolP, edata), jnp.bfloat16)
    out = jnp.maximum(out2[:_NW], out2[_NW:])
    return out.transpose(1, 0, 2).reshape(_N, _D).astype(jnp.float32)


def kernel(x, edge_index, edge_weight, enc_mask_token, Wp1, bp1, Ws1, Wn1, b1,
           Wp2, bp2, Ws2, Wn2, b2, W_e2d, W_np, b_np, gamma, beta):
    N, D = x.shape
    src = edge_index[0]
    dst = edge_index[1]
    perm = jax.random.permutation(jax.random.key(1), N)
    num_mask = int(0.2 * N)
    mask_nodes = perm[:num_mask]
    mask = jnp.zeros((N, 1), jnp.float32).at[mask_nodes, 0].set(1.0)

    pool1, self1 = _stage_a(mask, x, enc_mask_token, Wp1,
                            bp1.reshape(1, _D), Ws1)
    neigh1 = _segmax(pool1, src, dst, edge_weight)
    n_scores, pool2, self2 = _stage_b(
        self1, neigh1, Wn1, b1.reshape(1, _D), W_np, b_np.reshape(1, _NC),
        gamma.reshape(1, _NC), beta.reshape(1, _NC), W_e2d, Wp2,
        bp2.reshape(1, _D), Ws2)
    neigh2 = _segmax(pool2, src, dst, edge_weight)
    recon = _stage_c(self2, neigh2, Wn2, b2.reshape(1, _D))
    x_pred = recon[mask_nodes]
    x_true = x[mask_nodes]
    return (x_pred, x_true, n_scores)


# speedup vs baseline: 1.0133x; 1.0133x over previous
import functools

import jax
import jax.numpy as jnp
from jax import lax
from jax.experimental import pallas as pl
from jax.experimental.pallas import tpu as pltpu
from jax.experimental.pallas import tpu_sc as plsc

_N = 10000
_D = 128
_NC = 16
_BLK = 2000

_E = 320000
_NTILES = 32
_NW = _D // 2
_WPT = 4
_NSLOT = 8
_EC = 3200
_NCHUNK = _E // _EC
_CPH = _NCHUNK // 2
_GRP = _EC // 16
_BOTH_GE = 0x3F803F80


def _stage_a_body(mask_ref, x_ref, tok_ref, Wp_ref, bp_ref, Ws_ref,
                  pool_ref, self_ref):
    xm = jnp.where(mask_ref[...] > 0, tok_ref[...], x_ref[...])
    pool_ref[...] = jnp.maximum(
        jnp.dot(xm, Wp_ref[...], preferred_element_type=jnp.float32)
        + bp_ref[...], 0.0).astype(jnp.bfloat16)
    self_ref[...] = jnp.dot(xm, Ws_ref[...], preferred_element_type=jnp.float32)


def _stage_a(mask, x, tok, Wp, bp, Ws):
    grid = _N // _BLK
    row_spec = pl.BlockSpec((_BLK, _D), lambda i: (i, 0))
    full_mat = pl.BlockSpec((_D, _D), lambda i: (0, 0))
    full_vec = pl.BlockSpec((1, _D), lambda i: (0, 0))
    return pl.pallas_call(
        _stage_a_body,
        grid=(grid,),
        in_specs=[pl.BlockSpec((_BLK, 1), lambda i: (i, 0)),
                  row_spec, full_vec, full_mat, full_vec, full_mat],
        out_specs=[row_spec, row_spec],
        out_shape=[jax.ShapeDtypeStruct((_N, _D), jnp.bfloat16),
                   jax.ShapeDtypeStruct((_N, _D), jnp.float32)],
    )(mask, x, tok, Wp, bp, Ws)


def _stage_b_body(self1_ref, neigh1_ref, Wn1_ref, b1_ref, Wnp_ref, bnp_ref,
                  gamma_ref, beta_ref, We2d_ref, Wp2_ref, bp2_ref, Ws2_ref,
                  ns_ref, pool2_ref, self2_ref):
    enc = jnp.maximum(
        self1_ref[...]
        + jnp.dot(neigh1_ref[...].astype(jnp.float32), Wn1_ref[...],
                  preferred_element_type=jnp.float32)
        + b1_ref[...], 0.0)
    z = jnp.dot(enc, Wnp_ref[...], preferred_element_type=jnp.float32) \
        + bnp_ref[...]
    mu = jnp.mean(z, axis=-1, keepdims=True)
    var = jnp.mean((z - mu) ** 2, axis=-1, keepdims=True)
    ns_ref[...] = (z - mu) / jnp.sqrt(var + 1e-5) * gamma_ref[...] \
        + beta_ref[...]
    rep = jnp.dot(enc, We2d_ref[...], preferred_element_type=jnp.float32)
    pool2_ref[...] = jnp.maximum(
        jnp.dot(rep, Wp2_ref[...], preferred_element_type=jnp.float32)
        + bp2_ref[...], 0.0).astype(jnp.bfloat16)
    self2_ref[...] = jnp.dot(rep, Ws2_ref[...],
                             preferred_element_type=jnp.float32)


def _stage_b(self1, neigh1, Wn1, b1, Wnp, bnp, gamma, beta, We2d, Wp2, bp2,
             Ws2):
    grid = _N // _BLK
    row_spec = pl.BlockSpec((_BLK, _D), lambda i: (i, 0))
    full_mat = pl.BlockSpec((_D, _D), lambda i: (0, 0))
    full_vec = pl.BlockSpec((1, _D), lambda i: (0, 0))
    nc_mat = pl.BlockSpec((_D, _NC), lambda i: (0, 0))
    nc_vec = pl.BlockSpec((1, _NC), lambda i: (0, 0))
    return pl.pallas_call(
        _stage_b_body,
        grid=(grid,),
        in_specs=[row_spec, row_spec, full_mat, full_vec, nc_mat, nc_vec,
                  nc_vec, nc_vec, full_mat, full_mat, full_vec, full_mat],
        out_specs=[pl.BlockSpec((_BLK, _NC), lambda i: (i, 0)),
                   row_spec, row_spec],
        out_shape=[jax.ShapeDtypeStruct((_N, _NC), jnp.float32),
                   jax.ShapeDtypeStruct((_N, _D), jnp.bfloat16),
                   jax.ShapeDtypeStruct((_N, _D), jnp.float32)],
    )(self1, neigh1, Wn1, b1, Wnp, bnp, gamma, beta, We2d, Wp2, bp2, Ws2)


def _stage_c_body(self2_ref, neigh2_ref, Wn2_ref, b2_ref, recon_ref):
    recon_ref[...] = jnp.maximum(
        self2_ref[...]
        + jnp.dot(neigh2_ref[...].astype(jnp.float32), Wn2_ref[...],
                  preferred_element_type=jnp.float32)
        + b2_ref[...], 0.0)


def _stage_c(self2, neigh2, Wn2, b2):
    grid = _N // _BLK
    row_spec = pl.BlockSpec((_BLK, _D), lambda i: (i, 0))
    return pl.pallas_call(
        _stage_c_body,
        grid=(grid,),
        in_specs=[row_spec, row_spec,
                  pl.BlockSpec((_D, _D), lambda i: (0, 0)),
                  pl.BlockSpec((1, _D), lambda i: (0, 0))],
        out_specs=row_spec,
        out_shape=jax.ShapeDtypeStruct((_N, _D), jnp.float32),
    )(self2, neigh2, Wn2, b2)


_UNROLL = 8


def _sc_segmax_body(poolP_hbm, edata_hbm, out_hbm,
                    p0, p1, p2, p3, a0, a1, a2, a3,
                    ebuf0, ebuf1, chkb, sem0, sem1):
    pools = [p0, p1, p2, p3]
    accs = [a0, a1, a2, a3]
    ebufs = [ebuf0, ebuf1]
    sems = [sem0, sem1]
    core = lax.axis_index("c")
    sub = lax.axis_index("s")
    half = sub // _NSLOT
    fslot = sub % _NSLOT
    wbase = core * (_NSLOT * _WPT) + fslot * _WPT
    cbase = half * _CPH

    pltpu.async_copy(edata_hbm.at[cbase], ebuf0, sem0)
    pltpu.async_copy(edata_hbm.at[cbase + 1], ebuf1, sem1)

    for f in range(_WPT):
        pltpu.sync_copy(poolP_hbm.at[wbase + f], pools[f])

    zeros16 = jnp.zeros((16,), jnp.int32)

    def zbody(i, carry):
        for f in range(_WPT):
            accs[f][pl.ds(i * 16, 16)] = zeros16
        return carry

    lax.fori_loop(0, _N // 16, zbody, 0)

    def do_chunk(b):
        ebuf = ebufs[b]

        def grp_body(g, gcarry):
            iota = lax.iota(jnp.int32, 16)
            d16s, valss = [], []
            bad = None
            for j in range(_UNROLL):
                e = (g * _UNROLL + j) * 16
                s16 = ebuf[pl.ds(e, 16)]
                d16 = ebuf[pl.ds(_EC + e, 16)]
                w_bf = plsc.bitcast(ebuf[pl.ds(2 * _EC + e, 16)],
                                    jnp.bfloat16)
                vals = [plsc.bitcast(plsc.load_gather(pools[f], [s16]),
                                     jnp.bfloat16) * w_bf
                        for f in range(_WPT)]
                plsc.store_scatter(chkb, [d16], iota)
                rd = plsc.load_gather(chkb, [d16])
                miss = rd != iota
                bad = miss if bad is None else jnp.logical_or(bad, miss)
                d16s.append(d16)
                valss.append(vals)
            nodup = jnp.logical_not(jnp.any(bad))

            def fast(_):
                for j in range(_UNROLL):
                    for f in range(_WPT):
                        old = plsc.load_gather(accs[f], [d16s[j]])
                        new_bf = jnp.maximum(
                            plsc.bitcast(old, jnp.bfloat16), valss[j][f])
                        plsc.store_scatter(accs[f], [d16s[j]],
                                           plsc.bitcast(new_bf, jnp.int32))
                return 0

            def slow(_):
                pends = tuple([jnp.ones((16,), jnp.int32)] *
                              (_UNROLL * _WPT))

                def wcond(p):
                    anyp = p[0]
                    for q in p[1:]:
                        anyp = anyp | q
                    return jnp.any(anyp != 0)

                def wbody(p):
                    npend = []
                    for j in range(_UNROLL):
                        for f in range(_WPT):
                            k = j * _WPT + f
                            val = valss[j][f]
                            old = plsc.load_gather(accs[f], [d16s[j]])
                            new_bf = jnp.maximum(
                                plsc.bitcast(old, jnp.bfloat16), val)
                            plsc.store_scatter(
                                accs[f], [d16s[j]],
                                plsc.bitcast(new_bf, jnp.int32),
                                mask=(p[k] != 0))
                            chk = plsc.load_gather(accs[f], [d16s[j]])
                            ge = plsc.bitcast(chk, jnp.bfloat16) >= val
                            flag = plsc.bitcast(
                                jnp.where(ge, jnp.bfloat16(1.0),
                                          jnp.bfloat16(0.0)), jnp.int32)
                            npend.append(
                                (flag != _BOTH_GE).astype(jnp.int32))
                    return tuple(npend)

                lax.while_loop(wcond, wbody, pends)
                return 0

            lax.cond(nodup, fast, slow, 0)
            return gcarry

        lax.fori_loop(0, _GRP // _UNROLL, grp_body, 0)

    def chunk_pair(k, carry):
        for b in range(2):
            c = cbase + k * 2 + b
            pltpu.make_async_copy(edata_hbm.at[0], ebufs[b], sems[b]).wait()
            do_chunk(b)
            nxt = jnp.minimum(c + 2, cbase + _CPH - 1)
            pltpu.async_copy(edata_hbm.at[nxt], ebufs[b], sems[b])
        return carry

    lax.fori_loop(0, _CPH // 2, chunk_pair, 0)
    for b in range(2):
        pltpu.make_async_copy(edata_hbm.at[0], ebufs[b], sems[b]).wait()

    for f in range(_WPT):
        pltpu.sync_copy(accs[f], out_hbm.at[half * _NW + wbase + f])


def _segmax(pool, src, dst, ew):
    poolP = lax.bitcast_convert_type(
        pool.reshape(_N, _NW, 2), jnp.int32).T
    ewb = ew.astype(jnp.bfloat16)
    ewP = lax.bitcast_convert_type(jnp.stack([ewb, ewb], axis=-1), jnp.int32)
    edata = jnp.stack([src, dst, ewP]).reshape(3, _NCHUNK, _EC) \
        .transpose(1, 0, 2).reshape(_NCHUNK, 3 * _EC)

    mesh = plsc.VectorSubcoreMesh(core_axis_name="c", subcore_axis_name="s")
    kern = functools.partial(
        pl.kernel,
        mesh=mesh,
        compiler_params=pltpu.CompilerParams(needs_layout_passes=False),
        out_type=jax.ShapeDtypeStruct((2 * _NW, _N), jnp.int32),
        scratch_types=(
            [pltpu.VMEM((_N,), jnp.int32) for _ in range(2 * _WPT)]
            + [pltpu.VMEM((3 * _EC,), jnp.int32),
               pltpu.VMEM((3 * _EC,), jnp.int32),
               pltpu.VMEM((_N,), jnp.int32),
               pltpu.SemaphoreType.DMA, pltpu.SemaphoreType.DMA]
        ),
    )(_sc_segmax_body)
    h = lax.bitcast_convert_type(kern(poolP, edata).T, jnp.bfloat16)
    return jnp.maximum(h[:, :_NW], h[:, _NW:]).reshape(_N, _D)


def kernel(x, edge_index, edge_weight, enc_mask_token, Wp1, bp1, Ws1, Wn1, b1,
           Wp2, bp2, Ws2, Wn2, b2, W_e2d, W_np, b_np, gamma, beta):
    N, D = x.shape
    src = edge_index[0]
    dst = edge_index[1]
    perm = jax.random.permutation(jax.random.key(1), N)
    num_mask = int(0.2 * N)
    mask_nodes = perm[:num_mask]
    mask = jnp.zeros((N, 1), jnp.float32).at[mask_nodes, 0].set(1.0)

    pool1, self1 = _stage_a(mask, x, enc_mask_token, Wp1,
                            bp1.reshape(1, _D), Ws1)
    neigh1 = _segmax(pool1, src, dst, edge_weight)
    n_scores, pool2, self2 = _stage_b(
        self1, neigh1, Wn1, b1.reshape(1, _D), W_np, b_np.reshape(1, _NC),
        gamma.reshape(1, _NC), beta.reshape(1, _NC), W_e2d, Wp2,
        bp2.reshape(1, _D), Ws2)
    neigh2 = _segmax(pool2, src, dst, edge_weight)
    recon = _stage_c(self2, neigh2, Wn2, b2.reshape(1, _D))
    x_pred = recon[mask_nodes]
    x_true = x[mask_nodes]
    return (x_pred, x_true, n_scores)
